# baseline (speedup 1.0000x reference)
import functools

import jax
import jax.numpy as jnp
from jax import lax
from jax.experimental import pallas as pl
from jax.experimental.pallas import tpu as pltpu
from jax.experimental.pallas import tpu_sc as plsc
from jax._src.pallas import mpmd as _mpmd

B = 128
D = 768
POOL = 8192
PLEN = 8
HALF = (PLEN // 2) * D
KB = 4096
NKB = POOL // KB


def _pool_body(q_ref, k_ref, idx_ref, loss_ref, qn_scr, runmax_scr,
               runidx_scr, selcs_scr):
    i = pl.program_id(0)

    @pl.when(i == 0)
    def _init():
        q = q_ref[...]
        qn = jnp.sqrt(jnp.sum(q * q, axis=1, keepdims=True))
        qn_scr[...] = (q / jnp.maximum(qn, 1e-12)).astype(jnp.bfloat16)
        runmax_scr[...] = jnp.full((B, 1), -jnp.inf, dtype=jnp.float32)
        runidx_scr[...] = jnp.zeros((B, 1), jnp.int32)
        selcs_scr[...] = jnp.zeros((B, 1), jnp.float32)

    k = k_ref[...]
    kn = jnp.sqrt(jnp.sum(k * k, axis=1, keepdims=True))
    nk = (k / jnp.maximum(kn, 1e-12)).astype(jnp.bfloat16)
    cos = lax.dot_general(qn_scr[...], nk, (((1,), (1,)), ((), ())),
                          preferred_element_type=jnp.float32)
    bm = jnp.max(cos, axis=1, keepdims=True)
    cols = lax.broadcasted_iota(jnp.int32, (B, KB), 1)
    barg = jnp.min(jnp.where(cos == bm, cols, jnp.int32(2**30)),
                   axis=1, keepdims=True)
    colsum = jnp.sum(cos, axis=0, keepdims=True)
    scs = jnp.sum(jnp.where(cols == barg, colsum, 0.0),
                  axis=1, keepdims=True)
    upd = bm > runmax_scr[...]
    runidx_scr[...] = jnp.where(upd, barg + i * KB, runidx_scr[...])
    runmax_scr[...] = jnp.where(upd, bm, runmax_scr[...])
    selcs_scr[...] = jnp.where(upd, scs, selcs_scr[...])

    @pl.when(i == NKB - 1)
    def _fin():
        idx2d = runidx_scr[...].reshape(B // 4, 4)
        idx_ref[...] = jnp.concatenate(
            [idx2d, jnp.zeros((B // 4, 12), jnp.int32)], axis=1)
        loss_ref[...] = 1.0 - jnp.sum(selcs_scr[...], axis=(0, 1),
                                      keepdims=True) / (B * B)


def _topk_pool(q, keys):
    return pl.pallas_call(
        _pool_body,
        grid=(NKB,),
        in_specs=[
            pl.BlockSpec((B, D), lambda i: (0, 0)),
            pl.BlockSpec((KB, D), lambda i: (i, 0)),
        ],
        out_specs=[
            pl.BlockSpec((B // 4, 16), lambda i: (0, 0)),
            pl.BlockSpec((1, 1), lambda i: (0, 0)),
        ],
        out_shape=[
            jax.ShapeDtypeStruct((B // 4, 16), jnp.int32),
            jax.ShapeDtypeStruct((1, 1), jnp.float32),
        ],
        scratch_shapes=[
            pltpu.VMEM((B, D), jnp.bfloat16),
            pltpu.VMEM((B, 1), jnp.float32),
            pltpu.VMEM((B, 1), jnp.int32),
            pltpu.VMEM((B, 1), jnp.float32),
        ],
    )(q, keys)


def _sc_gather_chain(ep2, ep3, ep4, gp0, gp1, i2p, i3p, i4p):
    info = plsc.get_sparse_core_info()
    nc, ns = info.num_cores, info.num_subcores
    nw = nc * ns
    bpw = B // nw
    hp = PLEN // 2

    mesh = plsc.VectorSubcoreMesh(core_axis_name="c", subcore_axis_name="s")
    out_sds = jax.ShapeDtypeStruct((5, 2, B, hp, D), jnp.float32)

    def _worker():
        wid = lax.axis_index("s") * nc + lax.axis_index("c")
        return wid, wid * bpw

    def _body_a(ep_h, gp0_h, gp1_h, ixp_h, out_h, i16, r, g0, g1,
                semg, semw):
        wid, base = _worker()
        pltpu.sync_copy(ixp_h.at[wid], i16)
        c = pltpu.async_copy(ep_h.at[i16.at[pl.ds(0, bpw)]], r, semg)
        pltpu.sync_copy(gp0_h, g0)
        pltpu.sync_copy(gp1_h, g1)
        writes = []
        for li, g in ((0, g0), (1, g1)):
            for h in (0, 1):
                for j in range(bpw):
                    writes.append(pltpu.async_copy(
                        g.at[pl.ds(hp * h, hp)],
                        out_h.at[li, h, base + j], semw))
        c.wait()
        for h in (0, 1):
            writes.append(pltpu.async_copy(
                r.at[:, pl.ds(hp * h, hp), :],
                out_h.at[2, h, pl.ds(base, bpw)], semw))
        for w in writes:
            w.wait()

    def _mk_body_pool(li):
        def _body(ep_h, ixp_h, prev_h, out_h, i16, r, semg, semw):
            del prev_h
            wid, base = _worker()
            pltpu.sync_copy(ixp_h.at[wid], i16)
            c = pltpu.async_copy(ep_h.at[i16.at[pl.ds(0, bpw)]], r, semg)
            c.wait()
            w0 = pltpu.async_copy(r.at[:, pl.ds(0, hp), :],
                                  out_h.at[li, 0, pl.ds(base, bpw)], semw)
            w1 = pltpu.async_copy(r.at[:, pl.ds(hp, hp), :],
                                  out_h.at[li, 1, pl.ds(base, bpw)], semw)
            w0.wait()
            w1.wait()
        return _body

    call_a = _mpmd._mpmd_map(
        [(mesh, _body_a)],
        out_sds,
        scratch_types=[
            pltpu.VMEM((16,), jnp.int32),
            pltpu.VMEM((bpw, PLEN, D), jnp.float32),
            pltpu.VMEM((PLEN, D), jnp.float32),
            pltpu.VMEM((PLEN, D), jnp.float32),
            pltpu.SemaphoreType.DMA,
            pltpu.SemaphoreType.DMA,
        ],
        name="sc_bcast_pool2",
    )
    out = call_a(ep2, gp0, gp1, i2p)
    for li, ep, ixp in ((3, ep3, i3p), (4, ep4, i4p)):
        call_p = _mpmd._mpmd_map(
            [(mesh, _mk_body_pool(li))],
            out_sds,
            input_output_aliases={2: 0},
            scratch_types=[
                pltpu.VMEM((16,), jnp.int32),
                pltpu.VMEM((bpw, PLEN, D), jnp.float32),
                pltpu.SemaphoreType.DMA,
                pltpu.SemaphoreType.DMA,
            ],
            name=f"sc_pool{li}",
        )
        out = call_p(ep, ixp, out)
    return out


def kernel(query, g_p_0, g_p_1, e_p_2, e_p_3, e_p_4, e_k_2, e_k_3, e_k_4,
           train):
    del train
    idx2, loss2 = _topk_pool(query, e_k_2)
    idx3, loss3 = _topk_pool(query, e_k_3)
    idx4, loss4 = _topk_pool(query, e_k_4)

    prompts = _sc_gather_chain(e_p_2, e_p_3, e_p_4, g_p_0, g_p_1,
                               idx2, idx3, idx4)
    zero = jnp.zeros((2,), jnp.float32)
    losses = jnp.concatenate(
        [zero, loss2.reshape(1), loss3.reshape(1), loss4.reshape(1)])
    return prompts, losses


# speedup vs baseline: 1.0197x; 1.0197x over previous
import functools

import jax
import jax.numpy as jnp
from jax import lax
from jax.experimental import pallas as pl
from jax.experimental.pallas import tpu as pltpu
from jax.experimental.pallas import tpu_sc as plsc
from jax._src.pallas import mpmd as _mpmd

B = 128
D = 768
POOL = 8192
PLEN = 8
HALF = (PLEN // 2) * D
KB = 1024
NKB = POOL // (2 * KB)


def _pool_body(q_ref, ka_ref, kb_ref, idx_ref, loss_ref, qn_scr, runmax_scr,
               runidx_scr, selcs_scr):
    i = pl.program_id(0)

    @pl.when(i == 0)
    def _init():
        q = q_ref[...]
        qn = jnp.sqrt(jnp.sum(q * q, axis=1, keepdims=True))
        qn_scr[...] = (q / jnp.maximum(qn, 1e-12)).astype(jnp.bfloat16)
        runmax_scr[...] = jnp.full((B, 1), -jnp.inf, dtype=jnp.float32)
        runidx_scr[...] = jnp.zeros((B, 1), jnp.int32)
        selcs_scr[...] = jnp.zeros((B, 1), jnp.float32)

    def _block(k_ref, off):
        k = k_ref[...]
        kn = jnp.sqrt(jnp.sum(k * k, axis=1, keepdims=True))
        nk = (k / jnp.maximum(kn, 1e-12)).astype(jnp.bfloat16)
        cos = lax.dot_general(qn_scr[...], nk, (((1,), (1,)), ((), ())),
                              preferred_element_type=jnp.float32)
        bm = jnp.max(cos, axis=1, keepdims=True)
        cols = lax.broadcasted_iota(jnp.int32, (B, KB), 1)
        barg = jnp.min(jnp.where(cos == bm, cols, jnp.int32(2**30)),
                       axis=1, keepdims=True)
        colsum = jnp.sum(cos, axis=0, keepdims=True)
        scs = jnp.sum(jnp.where(cols == barg, colsum, 0.0),
                      axis=1, keepdims=True)
        upd = bm > runmax_scr[...]
        runidx_scr[...] = jnp.where(upd, barg + off, runidx_scr[...])
        runmax_scr[...] = jnp.where(upd, bm, runmax_scr[...])
        selcs_scr[...] = jnp.where(upd, scs, selcs_scr[...])

    _block(ka_ref, i * 2 * KB)
    _block(kb_ref, i * 2 * KB + KB)

    @pl.when(i == NKB - 1)
    def _fin():
        idx2d = runidx_scr[...].reshape(B // 4, 4)
        idx_ref[...] = jnp.concatenate(
            [idx2d, jnp.zeros((B // 4, 12), jnp.int32)], axis=1)
        loss_ref[...] = 1.0 - jnp.sum(selcs_scr[...], axis=(0, 1),
                                      keepdims=True) / (B * B)


def _topk_pool(q, keys):
    return pl.pallas_call(
        _pool_body,
        grid=(NKB,),
        in_specs=[
            pl.BlockSpec((B, D), lambda i: (0, 0)),
            pl.BlockSpec((KB, D), lambda i: (2 * i, 0)),
            pl.BlockSpec((KB, D), lambda i: (2 * i + 1, 0)),
        ],
        out_specs=[
            pl.BlockSpec((B // 4, 16), lambda i: (0, 0)),
            pl.BlockSpec((1, 1), lambda i: (0, 0)),
        ],
        out_shape=[
            jax.ShapeDtypeStruct((B // 4, 16), jnp.int32),
            jax.ShapeDtypeStruct((1, 1), jnp.float32),
        ],
        scratch_shapes=[
            pltpu.VMEM((B, D), jnp.bfloat16),
            pltpu.VMEM((B, 1), jnp.float32),
            pltpu.VMEM((B, 1), jnp.int32),
            pltpu.VMEM((B, 1), jnp.float32),
        ],
    )(q, keys, keys)


def _sc_gather_chain(ep2, ep3, ep4, gp0, gp1, i2p, i3p, i4p):
    info = plsc.get_sparse_core_info()
    nc, ns = info.num_cores, info.num_subcores
    nw = nc * ns
    bpw = B // nw
    hp = PLEN // 2

    mesh = plsc.VectorSubcoreMesh(core_axis_name="c", subcore_axis_name="s")
    out_sds = jax.ShapeDtypeStruct((5, 2, B, hp, D), jnp.float32)

    def _worker():
        wid = lax.axis_index("s") * nc + lax.axis_index("c")
        return wid, wid * bpw

    def _body_a(ep_h, gp0_h, gp1_h, ixp_h, out_h, i16, r, g0, g1,
                semg, semw):
        wid, base = _worker()
        pltpu.sync_copy(ixp_h.at[wid], i16)
        c = pltpu.async_copy(ep_h.at[i16.at[pl.ds(0, bpw)]], r, semg)
        pltpu.sync_copy(gp0_h, g0)
        pltpu.sync_copy(gp1_h, g1)
        writes = []
        for li, g in ((0, g0), (1, g1)):
            for h in (0, 1):
                for j in range(bpw):
                    writes.append(pltpu.async_copy(
                        g.at[pl.ds(hp * h, hp)],
                        out_h.at[li, h, base + j], semw))
        c.wait()
        for h in (0, 1):
            writes.append(pltpu.async_copy(
                r.at[:, pl.ds(hp * h, hp), :],
                out_h.at[2, h, pl.ds(base, bpw)], semw))
        for w in writes:
            w.wait()

    def _mk_body_pool(li):
        def _body(ep_h, ixp_h, prev_h, out_h, i16, r, semg, semw):
            del prev_h
            wid, base = _worker()
            pltpu.sync_copy(ixp_h.at[wid], i16)
            c = pltpu.async_copy(ep_h.at[i16.at[pl.ds(0, bpw)]], r, semg)
            c.wait()
            w0 = pltpu.async_copy(r.at[:, pl.ds(0, hp), :],
                                  out_h.at[li, 0, pl.ds(base, bpw)], semw)
            w1 = pltpu.async_copy(r.at[:, pl.ds(hp, hp), :],
                                  out_h.at[li, 1, pl.ds(base, bpw)], semw)
            w0.wait()
            w1.wait()
        return _body

    call_a = _mpmd._mpmd_map(
        [(mesh, _body_a)],
        out_sds,
        scratch_types=[
            pltpu.VMEM((16,), jnp.int32),
            pltpu.VMEM((bpw, PLEN, D), jnp.float32),
            pltpu.VMEM((PLEN, D), jnp.float32),
            pltpu.VMEM((PLEN, D), jnp.float32),
            pltpu.SemaphoreType.DMA,
            pltpu.SemaphoreType.DMA,
        ],
        name="sc_bcast_pool2",
    )
    out = call_a(ep2, gp0, gp1, i2p)
    for li, ep, ixp in ((3, ep3, i3p), (4, ep4, i4p)):
        call_p = _mpmd._mpmd_map(
            [(mesh, _mk_body_pool(li))],
            out_sds,
            input_output_aliases={2: 0},
            scratch_types=[
                pltpu.VMEM((16,), jnp.int32),
                pltpu.VMEM((bpw, PLEN, D), jnp.float32),
                pltpu.SemaphoreType.DMA,
                pltpu.SemaphoreType.DMA,
            ],
            name=f"sc_pool{li}",
        )
        out = call_p(ep, ixp, out)
    return out


def kernel(query, g_p_0, g_p_1, e_p_2, e_p_3, e_p_4, e_k_2, e_k_3, e_k_4,
           train):
    del train
    idx2, loss2 = _topk_pool(query, e_k_2)
    idx3, loss3 = _topk_pool(query, e_k_3)
    idx4, loss4 = _topk_pool(query, e_k_4)

    prompts = _sc_gather_chain(e_p_2, e_p_3, e_p_4, g_p_0, g_p_1,
                               idx2, idx3, idx4)
    zero = jnp.zeros((2,), jnp.float32)
    losses = jnp.concatenate(
        [zero, loss2.reshape(1), loss3.reshape(1), loss4.reshape(1)])
    return prompts, losses


# speedup vs baseline: 1.0463x; 1.0261x over previous
import functools

import jax
import jax.numpy as jnp
from jax import lax
from jax.experimental import pallas as pl
from jax.experimental.pallas import tpu as pltpu
from jax.experimental.pallas import tpu_sc as plsc
from jax._src.pallas import mpmd as _mpmd

B = 128
D = 768
POOL = 8192
PLEN = 8
HALF = (PLEN // 2) * D
KB = 2048
NKB = POOL // KB


def _pool_body(q_ref, k_ref, idx_ref, loss_ref, qn_scr, runmax_scr,
               runidx_scr, selcs_scr):
    i = pl.program_id(0)

    @pl.when(i == 0)
    def _init():
        q = q_ref[...]
        qn = jnp.sqrt(jnp.sum(q * q, axis=1, keepdims=True))
        qn_scr[...] = (q / jnp.maximum(qn, 1e-12)).astype(jnp.bfloat16)
        runmax_scr[...] = jnp.full((B, 1), -jnp.inf, dtype=jnp.float32)
        runidx_scr[...] = jnp.zeros((B, 1), jnp.int32)
        selcs_scr[...] = jnp.zeros((B, 1), jnp.float32)

    def _block(k_ref, off):
        k = k_ref[...]
        kn = jnp.sqrt(jnp.sum(k * k, axis=1, keepdims=True))
        nk = (k / jnp.maximum(kn, 1e-12)).astype(jnp.bfloat16)
        cos = lax.dot_general(qn_scr[...], nk, (((1,), (1,)), ((), ())),
                              preferred_element_type=jnp.float32)
        bm = jnp.max(cos, axis=1, keepdims=True)
        cols = lax.broadcasted_iota(jnp.int32, (B, KB), 1)
        barg = jnp.min(jnp.where(cos == bm, cols, jnp.int32(2**30)),
                       axis=1, keepdims=True)
        colsum = jnp.sum(cos, axis=0, keepdims=True)
        scs = jnp.sum(jnp.where(cols == barg, colsum, 0.0),
                      axis=1, keepdims=True)
        upd = bm > runmax_scr[...]
        runidx_scr[...] = jnp.where(upd, barg + off, runidx_scr[...])
        runmax_scr[...] = jnp.where(upd, bm, runmax_scr[...])
        selcs_scr[...] = jnp.where(upd, scs, selcs_scr[...])

    _block(k_ref, i * KB)

    @pl.when(i == NKB - 1)
    def _fin():
        idx2d = runidx_scr[...].reshape(B // 4, 4)
        idx_ref[...] = jnp.concatenate(
            [idx2d, jnp.zeros((B // 4, 12), jnp.int32)], axis=1)
        loss_ref[...] = 1.0 - jnp.sum(selcs_scr[...], axis=(0, 1),
                                      keepdims=True) / (B * B)


def _topk_pool(q, keys):
    return pl.pallas_call(
        _pool_body,
        grid=(NKB,),
        in_specs=[
            pl.BlockSpec((B, D), lambda i: (0, 0)),
            pl.BlockSpec((KB, D), lambda i: (i, 0)),
        ],
        out_specs=[
            pl.BlockSpec((B // 4, 16), lambda i: (0, 0)),
            pl.BlockSpec((1, 1), lambda i: (0, 0)),
        ],
        out_shape=[
            jax.ShapeDtypeStruct((B // 4, 16), jnp.int32),
            jax.ShapeDtypeStruct((1, 1), jnp.float32),
        ],
        scratch_shapes=[
            pltpu.VMEM((B, D), jnp.bfloat16),
            pltpu.VMEM((B, 1), jnp.float32),
            pltpu.VMEM((B, 1), jnp.int32),
            pltpu.VMEM((B, 1), jnp.float32),
        ],
    )(q, keys)


def _sc_gather_chain(ep2, ep3, ep4, gp0, gp1, i2p, i3p, i4p):
    info = plsc.get_sparse_core_info()
    nc, ns = info.num_cores, info.num_subcores
    nw = nc * ns
    bpw = B // nw
    hp = PLEN // 2

    mesh = plsc.VectorSubcoreMesh(core_axis_name="c", subcore_axis_name="s")
    out_sds = jax.ShapeDtypeStruct((5, 2, B, hp, D), jnp.float32)

    def _worker():
        wid = lax.axis_index("s") * nc + lax.axis_index("c")
        return wid, wid * bpw

    def _body_a(ep_h, gp0_h, gp1_h, ixp_h, out_h, i16, r, g0, g1,
                semg, semw):
        wid, base = _worker()
        pltpu.sync_copy(ixp_h.at[wid], i16)
        c = pltpu.async_copy(ep_h.at[i16.at[pl.ds(0, bpw)]], r, semg)
        pltpu.sync_copy(gp0_h, g0)
        pltpu.sync_copy(gp1_h, g1)
        writes = []
        for li, g in ((0, g0), (1, g1)):
            for h in (0, 1):
                for j in range(bpw):
                    writes.append(pltpu.async_copy(
                        g.at[pl.ds(hp * h, hp)],
                        out_h.at[li, h, base + j], semw))
        c.wait()
        for h in (0, 1):
            writes.append(pltpu.async_copy(
                r.at[:, pl.ds(hp * h, hp), :],
                out_h.at[2, h, pl.ds(base, bpw)], semw))
        for w in writes:
            w.wait()

    def _mk_body_pool(li):
        def _body(ep_h, ixp_h, prev_h, out_h, i16, r, semg, semw):
            del prev_h
            wid, base = _worker()
            pltpu.sync_copy(ixp_h.at[wid], i16)
            c = pltpu.async_copy(ep_h.at[i16.at[pl.ds(0, bpw)]], r, semg)
            c.wait()
            w0 = pltpu.async_copy(r.at[:, pl.ds(0, hp), :],
                                  out_h.at[li, 0, pl.ds(base, bpw)], semw)
            w1 = pltpu.async_copy(r.at[:, pl.ds(hp, hp), :],
                                  out_h.at[li, 1, pl.ds(base, bpw)], semw)
            w0.wait()
            w1.wait()
        return _body

    call_a = _mpmd._mpmd_map(
        [(mesh, _body_a)],
        out_sds,
        scratch_types=[
            pltpu.VMEM((16,), jnp.int32),
            pltpu.VMEM((bpw, PLEN, D), jnp.float32),
            pltpu.VMEM((PLEN, D), jnp.float32),
            pltpu.VMEM((PLEN, D), jnp.float32),
            pltpu.SemaphoreType.DMA,
            pltpu.SemaphoreType.DMA,
        ],
        name="sc_bcast_pool2",
    )
    out = call_a(ep2, gp0, gp1, i2p)
    for li, ep, ixp in ((3, ep3, i3p), (4, ep4, i4p)):
        call_p = _mpmd._mpmd_map(
            [(mesh, _mk_body_pool(li))],
            out_sds,
            input_output_aliases={2: 0},
            scratch_types=[
                pltpu.VMEM((16,), jnp.int32),
                pltpu.VMEM((bpw, PLEN, D), jnp.float32),
                pltpu.SemaphoreType.DMA,
                pltpu.SemaphoreType.DMA,
            ],
            name=f"sc_pool{li}",
        )
        out = call_p(ep, ixp, out)
    return out


def kernel(query, g_p_0, g_p_1, e_p_2, e_p_3, e_p_4, e_k_2, e_k_3, e_k_4,
           train):
    del train
    idx2, loss2 = _topk_pool(query, e_k_2)
    idx3, loss3 = _topk_pool(query, e_k_3)
    idx4, loss4 = _topk_pool(query, e_k_4)

    prompts = _sc_gather_chain(e_p_2, e_p_3, e_p_4, g_p_0, g_p_1,
                               idx2, idx3, idx4)
    zero = jnp.zeros((2,), jnp.float32)
    losses = jnp.concatenate(
        [zero, loss2.reshape(1), loss3.reshape(1), loss4.reshape(1)])
    return prompts, losses
